# baseline (speedup 1.0000x reference)
import functools

import jax
import jax.numpy as jnp
from jax import lax
from jax.experimental import pallas as pl
from jax.experimental.pallas import tpu as pltpu
from jax.experimental.pallas import tpu_sc as plsc

B = 262144
K = 64

BSC = 0
BT = B - BSC

CB = 16384
TCBL = 128

NW = 32
CBS = 256
TCB = CBS // 128


def _tc_body(gu_ref, gi_ref, bu_ref, bi_ref, mu_ref, out_ref):
    z = gu_ref[...] * gi_ref[...]
    out_ref[...] = jnp.sum(z, axis=0) + bu_ref[...] + bi_ref[...] + mu_ref[0, 0]


def _tc_call(gut, git, buf, bif, Mu, bt):
    return pl.pallas_call(
        _tc_body,
        grid=(bt // CB,),
        compiler_params=pltpu.CompilerParams(
            dimension_semantics=("parallel",),
        ),
        in_specs=[
            pl.BlockSpec((K, CB), lambda i: (0, i)),
            pl.BlockSpec((K, CB), lambda i: (0, i)),
            pl.BlockSpec((CB,), lambda i: (i,)),
            pl.BlockSpec((CB,), lambda i: (i,)),
            pl.BlockSpec((1, 1), lambda i: (0, 0)),
        ],
        out_specs=pl.BlockSpec((CB,), lambda i: (i,)),
        out_shape=jax.ShapeDtypeStruct((bt,), jnp.float32),
    )(gut, git, buf, bif, Mu)


def _tc_body2(gu_ref, gi_ref, bu_ref, bi_ref, mu_ref, out_ref):
    j = pl.program_id(1)
    s = jnp.sum(gu_ref[...] * gi_ref[...], axis=0)

    @pl.when(j == 0)
    def _():
        out_ref[...] = s + bu_ref[...] + bi_ref[...] + mu_ref[0, 0]

    @pl.when(j > 0)
    def _():
        out_ref[...] = out_ref[...] + s


def _tc_call2(gut, git, buf, bif, Mu, bt):
    return pl.pallas_call(
        _tc_body2,
        grid=(bt // CB, 8),
        compiler_params=pltpu.CompilerParams(
            dimension_semantics=("parallel", "arbitrary"),
        ),
        in_specs=[
            pl.BlockSpec((8, CB), lambda i, j: (j, i)),
            pl.BlockSpec((8, CB), lambda i, j: (j, i)),
            pl.BlockSpec((CB,), lambda i, j: (i,)),
            pl.BlockSpec((CB,), lambda i, j: (i,)),
            pl.BlockSpec((1, 1), lambda i, j: (0, 0)),
        ],
        out_specs=pl.BlockSpec((CB,), lambda i, j: (i,)),
        out_shape=jax.ShapeDtypeStruct((bt,), jnp.float32),
    )(gut, git, buf, bif, Mu)


def _make_sc(bsc, col0):
    cols_pw = bsc // NW
    nchunks = cols_pw // CBS
    mesh = plsc.VectorSubcoreMesh(core_axis_name="c", subcore_axis_name="s")

    @functools.partial(
        pl.kernel,
        out_type=jax.ShapeDtypeStruct((bsc,), jnp.float32),
        mesh=mesh,
        scratch_types=[
            pltpu.VMEM((8, TCB, 8, 128), jnp.float32),
            pltpu.VMEM((8, TCB, 8, 128), jnp.float32),
            pltpu.VMEM((cols_pw,), jnp.float32),
            pltpu.VMEM((cols_pw,), jnp.float32),
            pltpu.VMEM((cols_pw,), jnp.float32),
            pltpu.VMEM((16,), jnp.float32),
        ],
    )
    def sck(gu4, gi4, buf, bif, mu16, out_hbm, gu_v, gi_v, bu_v, bi_v, out_v, mu_v):
        wid = lax.axis_index("s") * 2 + lax.axis_index("c")
        base = col0 + wid * cols_pw
        pltpu.sync_copy(mu16, mu_v)
        pltpu.sync_copy(buf.at[pl.ds(base, cols_pw)], bu_v)
        pltpu.sync_copy(bif.at[pl.ds(base, cols_pw)], bi_v)

        def chunk(j, _):
            col = base + j * CBS
            pltpu.sync_copy(gu4.at[:, pl.ds(col // 128, TCB)], gu_v)
            pltpu.sync_copy(gi4.at[:, pl.ds(col // 128, TCB)], gi_v)

            def grp(g, _):
                lo = (g % 8) * 16
                tc = g // 8
                acc = mu_v[...]
                for kk in range(K):
                    tr, s = kk // 8, kk % 8
                    acc = acc + gu_v[tr, tc, s, pl.ds(lo, 16)] * gi_v[tr, tc, s, pl.ds(lo, 16)]
                o = j * CBS + g * 16
                out_v[pl.ds(o, 16)] = acc + bu_v[pl.ds(o, 16)] + bi_v[pl.ds(o, 16)]
                return 0

            lax.fori_loop(0, CBS // 16, grp, 0)
            return 0

        lax.fori_loop(0, nchunks, chunk, 0)
        pltpu.sync_copy(out_v, out_hbm.at[pl.ds(base - col0, cols_pw)])

    return sck


def kernel(gu, gi, bu, bi, Mu):
    gut = gu.T
    git = gi.T
    gu4 = gut.reshape(8, 8, B // 128, 128).transpose(0, 2, 1, 3)
    gi4 = git.reshape(8, 8, B // 128, 128).transpose(0, 2, 1, 3)
    buf = bu.reshape(B)
    bif = bi.reshape(B)
    mu16 = jnp.broadcast_to(Mu.reshape(1), (16,))

    parts = []
    if BSC > 0:
        sc_out = _make_sc(BSC, BT)(gu4, gi4, buf, bif, mu16)
    if BT > 0:
        parts.append(_tc_call2(gut, git, buf[:BT], bif[:BT], Mu, BT))
    if BSC > 0:
        parts.append(sc_out)
    return parts[0] if len(parts) == 1 else jnp.concatenate(parts)


# speedup vs baseline: 2.4065x; 2.4065x over previous
import functools

import jax
import jax.numpy as jnp
from jax import lax
from jax.experimental import pallas as pl
from jax.experimental.pallas import tpu as pltpu
from jax.experimental.pallas import tpu_sc as plsc

B = 262144
K = 64

BSC = 0
BT = B - BSC

CB = 16384
TCBL = 128

NW = 32
CBS = 256
TCB = CBS // 128


def _tc_body(gu_ref, gi_ref, bu_ref, bi_ref, mu_ref, out_ref):
    z = gu_ref[...] * gi_ref[...]
    out_ref[...] = jnp.sum(z, axis=0) + bu_ref[...] + bi_ref[...] + mu_ref[0, 0]


def _tc_call(gut, git, buf, bif, Mu, bt):
    return pl.pallas_call(
        _tc_body,
        grid=(bt // CB,),
        compiler_params=pltpu.CompilerParams(
            dimension_semantics=("parallel",),
        ),
        in_specs=[
            pl.BlockSpec((K, CB), lambda i: (0, i)),
            pl.BlockSpec((K, CB), lambda i: (0, i)),
            pl.BlockSpec((CB,), lambda i: (i,)),
            pl.BlockSpec((CB,), lambda i: (i,)),
            pl.BlockSpec((1, 1), lambda i: (0, 0)),
        ],
        out_specs=pl.BlockSpec((CB,), lambda i: (i,)),
        out_shape=jax.ShapeDtypeStruct((bt,), jnp.float32),
    )(gut, git, buf, bif, Mu)


def _make_sc(bsc, col0):
    cols_pw = bsc // NW
    nchunks = cols_pw // CBS
    mesh = plsc.VectorSubcoreMesh(core_axis_name="c", subcore_axis_name="s")

    @functools.partial(
        pl.kernel,
        out_type=jax.ShapeDtypeStruct((bsc,), jnp.float32),
        mesh=mesh,
        scratch_types=[
            pltpu.VMEM((8, TCB, 8, 128), jnp.float32),
            pltpu.VMEM((8, TCB, 8, 128), jnp.float32),
            pltpu.VMEM((cols_pw,), jnp.float32),
            pltpu.VMEM((cols_pw,), jnp.float32),
            pltpu.VMEM((cols_pw,), jnp.float32),
            pltpu.VMEM((16,), jnp.float32),
        ],
    )
    def sck(gu4, gi4, buf, bif, mu16, out_hbm, gu_v, gi_v, bu_v, bi_v, out_v, mu_v):
        wid = lax.axis_index("s") * 2 + lax.axis_index("c")
        base = col0 + wid * cols_pw
        pltpu.sync_copy(mu16, mu_v)
        pltpu.sync_copy(buf.at[pl.ds(base, cols_pw)], bu_v)
        pltpu.sync_copy(bif.at[pl.ds(base, cols_pw)], bi_v)

        def chunk(j, _):
            col = base + j * CBS
            pltpu.sync_copy(gu4.at[:, pl.ds(col // 128, TCB)], gu_v)
            pltpu.sync_copy(gi4.at[:, pl.ds(col // 128, TCB)], gi_v)

            def grp(g, _):
                lo = (g % 8) * 16
                tc = g // 8
                acc = mu_v[...]
                for kk in range(K):
                    tr, s = kk // 8, kk % 8
                    acc = acc + gu_v[tr, tc, s, pl.ds(lo, 16)] * gi_v[tr, tc, s, pl.ds(lo, 16)]
                o = j * CBS + g * 16
                out_v[pl.ds(o, 16)] = acc + bu_v[pl.ds(o, 16)] + bi_v[pl.ds(o, 16)]
                return 0

            lax.fori_loop(0, CBS // 16, grp, 0)
            return 0

        lax.fori_loop(0, nchunks, chunk, 0)
        pltpu.sync_copy(out_v, out_hbm.at[pl.ds(base - col0, cols_pw)])

    return sck


def kernel(gu, gi, bu, bi, Mu):
    gut = gu.T
    git = gi.T
    gu4 = gut.reshape(8, 8, B // 128, 128).transpose(0, 2, 1, 3)
    gi4 = git.reshape(8, 8, B // 128, 128).transpose(0, 2, 1, 3)
    buf = bu.reshape(B)
    bif = bi.reshape(B)
    mu16 = jnp.broadcast_to(Mu.reshape(1), (16,))

    parts = []
    if BSC > 0:
        sc_out = _make_sc(BSC, BT)(gu4, gi4, buf, bif, mu16)
    if BT > 0:
        parts.append(_tc_call(gut, git, buf[:BT], bif[:BT], Mu, BT))
    if BSC > 0:
        parts.append(sc_out)
    return parts[0] if len(parts) == 1 else jnp.concatenate(parts)


# speedup vs baseline: 2.4443x; 1.0157x over previous
import functools

import jax
import jax.numpy as jnp
from jax import lax
from jax.experimental import pallas as pl
from jax.experimental.pallas import tpu as pltpu
from jax.experimental.pallas import tpu_sc as plsc

B = 262144
K = 64

BSC = 0
BT = B - BSC

CB = 16384
TCBL = 128

NW = 32
CBS = 256
TCB = CBS // 128


SUB = 2048


def _tc_body(gu_ref, gi_ref, bu_ref, bi_ref, mu_ref, out_ref):
    for c in range(0, CB, SUB):
        cs = pl.ds(c, SUB)
        acc = gu_ref[0:8, cs] * gi_ref[0:8, cs]
        for r in range(8, K, 8):
            acc = acc + gu_ref[r:r + 8, cs] * gi_ref[r:r + 8, cs]
        out_ref[cs] = jnp.sum(acc, axis=0) + bu_ref[cs] + bi_ref[cs] + mu_ref[0, 0]


def _tc_call(gut, git, buf, bif, Mu, bt):
    return pl.pallas_call(
        _tc_body,
        grid=(bt // CB,),
        compiler_params=pltpu.CompilerParams(
            dimension_semantics=("parallel",),
        ),
        in_specs=[
            pl.BlockSpec((K, CB), lambda i: (0, i)),
            pl.BlockSpec((K, CB), lambda i: (0, i)),
            pl.BlockSpec((CB,), lambda i: (i,)),
            pl.BlockSpec((CB,), lambda i: (i,)),
            pl.BlockSpec((1, 1), lambda i: (0, 0)),
        ],
        out_specs=pl.BlockSpec((CB,), lambda i: (i,)),
        out_shape=jax.ShapeDtypeStruct((bt,), jnp.float32),
    )(gut, git, buf, bif, Mu)


def _make_sc(bsc, col0):
    cols_pw = bsc // NW
    nchunks = cols_pw // CBS
    mesh = plsc.VectorSubcoreMesh(core_axis_name="c", subcore_axis_name="s")

    @functools.partial(
        pl.kernel,
        out_type=jax.ShapeDtypeStruct((bsc,), jnp.float32),
        mesh=mesh,
        scratch_types=[
            pltpu.VMEM((8, TCB, 8, 128), jnp.float32),
            pltpu.VMEM((8, TCB, 8, 128), jnp.float32),
            pltpu.VMEM((cols_pw,), jnp.float32),
            pltpu.VMEM((cols_pw,), jnp.float32),
            pltpu.VMEM((cols_pw,), jnp.float32),
            pltpu.VMEM((16,), jnp.float32),
        ],
    )
    def sck(gu4, gi4, buf, bif, mu16, out_hbm, gu_v, gi_v, bu_v, bi_v, out_v, mu_v):
        wid = lax.axis_index("s") * 2 + lax.axis_index("c")
        base = col0 + wid * cols_pw
        pltpu.sync_copy(mu16, mu_v)
        pltpu.sync_copy(buf.at[pl.ds(base, cols_pw)], bu_v)
        pltpu.sync_copy(bif.at[pl.ds(base, cols_pw)], bi_v)

        def chunk(j, _):
            col = base + j * CBS
            pltpu.sync_copy(gu4.at[:, pl.ds(col // 128, TCB)], gu_v)
            pltpu.sync_copy(gi4.at[:, pl.ds(col // 128, TCB)], gi_v)

            def grp(g, _):
                lo = (g % 8) * 16
                tc = g // 8
                acc = mu_v[...]
                for kk in range(K):
                    tr, s = kk // 8, kk % 8
                    acc = acc + gu_v[tr, tc, s, pl.ds(lo, 16)] * gi_v[tr, tc, s, pl.ds(lo, 16)]
                o = j * CBS + g * 16
                out_v[pl.ds(o, 16)] = acc + bu_v[pl.ds(o, 16)] + bi_v[pl.ds(o, 16)]
                return 0

            lax.fori_loop(0, CBS // 16, grp, 0)
            return 0

        lax.fori_loop(0, nchunks, chunk, 0)
        pltpu.sync_copy(out_v, out_hbm.at[pl.ds(base - col0, cols_pw)])

    return sck


def kernel(gu, gi, bu, bi, Mu):
    gut = gu.T
    git = gi.T
    gu4 = gut.reshape(8, 8, B // 128, 128).transpose(0, 2, 1, 3)
    gi4 = git.reshape(8, 8, B // 128, 128).transpose(0, 2, 1, 3)
    buf = bu.reshape(B)
    bif = bi.reshape(B)
    mu16 = jnp.broadcast_to(Mu.reshape(1), (16,))

    parts = []
    if BSC > 0:
        sc_out = _make_sc(BSC, BT)(gu4, gi4, buf, bif, mu16)
    if BT > 0:
        parts.append(_tc_call(gut, git, buf[:BT], bif[:BT], Mu, BT))
    if BSC > 0:
        parts.append(sc_out)
    return parts[0] if len(parts) == 1 else jnp.concatenate(parts)


# speedup vs baseline: 2.4519x; 1.0031x over previous
import functools

import jax
import jax.numpy as jnp
from jax import lax
from jax.experimental import pallas as pl
from jax.experimental.pallas import tpu as pltpu
from jax.experimental.pallas import tpu_sc as plsc

B = 262144
K = 64

BSC = 0
BT = B - BSC

CB = 16384
TCBL = 128

NW = 32
CBS = 256
TCB = CBS // 128


SUB = 1024


def _tc_body(gu_ref, gi_ref, bu_ref, bi_ref, mu_ref, out_ref):
    for c in range(0, CB, SUB):
        cs = pl.ds(c, SUB)
        acc = gu_ref[0:8, cs] * gi_ref[0:8, cs]
        for r in range(8, K, 8):
            acc = acc + gu_ref[r:r + 8, cs] * gi_ref[r:r + 8, cs]
        out_ref[cs] = jnp.sum(acc, axis=0) + bu_ref[cs] + bi_ref[cs] + mu_ref[0, 0]


def _tc_call(gut, git, buf, bif, Mu, bt):
    return pl.pallas_call(
        _tc_body,
        grid=(bt // CB,),
        compiler_params=pltpu.CompilerParams(
            dimension_semantics=("parallel",),
        ),
        in_specs=[
            pl.BlockSpec((K, CB), lambda i: (0, i)),
            pl.BlockSpec((K, CB), lambda i: (0, i)),
            pl.BlockSpec((CB,), lambda i: (i,)),
            pl.BlockSpec((CB,), lambda i: (i,)),
            pl.BlockSpec((1, 1), lambda i: (0, 0)),
        ],
        out_specs=pl.BlockSpec((CB,), lambda i: (i,)),
        out_shape=jax.ShapeDtypeStruct((bt,), jnp.float32),
    )(gut, git, buf, bif, Mu)


def _make_sc(bsc, col0):
    cols_pw = bsc // NW
    nchunks = cols_pw // CBS
    mesh = plsc.VectorSubcoreMesh(core_axis_name="c", subcore_axis_name="s")

    @functools.partial(
        pl.kernel,
        out_type=jax.ShapeDtypeStruct((bsc,), jnp.float32),
        mesh=mesh,
        scratch_types=[
            pltpu.VMEM((8, TCB, 8, 128), jnp.float32),
            pltpu.VMEM((8, TCB, 8, 128), jnp.float32),
            pltpu.VMEM((cols_pw,), jnp.float32),
            pltpu.VMEM((cols_pw,), jnp.float32),
            pltpu.VMEM((cols_pw,), jnp.float32),
            pltpu.VMEM((16,), jnp.float32),
        ],
    )
    def sck(gu4, gi4, buf, bif, mu16, out_hbm, gu_v, gi_v, bu_v, bi_v, out_v, mu_v):
        wid = lax.axis_index("s") * 2 + lax.axis_index("c")
        base = col0 + wid * cols_pw
        pltpu.sync_copy(mu16, mu_v)
        pltpu.sync_copy(buf.at[pl.ds(base, cols_pw)], bu_v)
        pltpu.sync_copy(bif.at[pl.ds(base, cols_pw)], bi_v)

        def chunk(j, _):
            col = base + j * CBS
            pltpu.sync_copy(gu4.at[:, pl.ds(col // 128, TCB)], gu_v)
            pltpu.sync_copy(gi4.at[:, pl.ds(col // 128, TCB)], gi_v)

            def grp(g, _):
                lo = (g % 8) * 16
                tc = g // 8
                acc = mu_v[...]
                for kk in range(K):
                    tr, s = kk // 8, kk % 8
                    acc = acc + gu_v[tr, tc, s, pl.ds(lo, 16)] * gi_v[tr, tc, s, pl.ds(lo, 16)]
                o = j * CBS + g * 16
                out_v[pl.ds(o, 16)] = acc + bu_v[pl.ds(o, 16)] + bi_v[pl.ds(o, 16)]
                return 0

            lax.fori_loop(0, CBS // 16, grp, 0)
            return 0

        lax.fori_loop(0, nchunks, chunk, 0)
        pltpu.sync_copy(out_v, out_hbm.at[pl.ds(base - col0, cols_pw)])

    return sck


def kernel(gu, gi, bu, bi, Mu):
    gut = gu.T
    git = gi.T
    gu4 = gut.reshape(8, 8, B // 128, 128).transpose(0, 2, 1, 3)
    gi4 = git.reshape(8, 8, B // 128, 128).transpose(0, 2, 1, 3)
    buf = bu.reshape(B)
    bif = bi.reshape(B)
    mu16 = jnp.broadcast_to(Mu.reshape(1), (16,))

    parts = []
    if BSC > 0:
        sc_out = _make_sc(BSC, BT)(gu4, gi4, buf, bif, mu16)
    if BT > 0:
        parts.append(_tc_call(gut, git, buf[:BT], bif[:BT], Mu, BT))
    if BSC > 0:
        parts.append(sc_out)
    return parts[0] if len(parts) == 1 else jnp.concatenate(parts)
